# baseline (speedup 1.0000x reference)
import functools

import jax
import jax.numpy as jnp
from jax import lax
from jax.experimental import pallas as pl
from jax.experimental.pallas import tpu as pltpu
from jax.experimental.pallas import tpu_sc as plsc

NUM_GATES = 1000
BATCH = 16384
DIM = 128
LANES = 16

NC = 2
NS = 16
NW = NC * NS

B_PER_W = BATCH // NW
CHUNK = 128
NCHUNK = B_PER_W // CHUNK


def _sc_gate_mul(x, idx, gates):
    mesh = plsc.VectorSubcoreMesh(core_axis_name="c", subcore_axis_name="s")

    @functools.partial(
        pl.kernel,
        mesh=mesh,
        out_type=jax.ShapeDtypeStruct((BATCH, DIM), jnp.float32),
        scratch_types=[
            pltpu.VMEM((CHUNK,), jnp.int32),
            pltpu.VMEM((CHUNK, DIM), jnp.float32),
            pltpu.VMEM((CHUNK, DIM), jnp.float32),
            pltpu.SemaphoreType.DMA,
        ],
    )
    def k(x_hbm, idx_hbm, gates_hbm, out_hbm, idx_v, g_v, x_v, sem):
        wid = lax.axis_index("s") * NC + lax.axis_index("c")
        base = wid * B_PER_W

        def chunk_body(c, carry):
            b0 = base + c * CHUNK
            pltpu.sync_copy(idx_hbm.at[pl.ds(b0, CHUNK)], idx_v)
            gather = pltpu.async_copy(gates_hbm.at[idx_v], g_v, sem)
            pltpu.sync_copy(x_hbm.at[pl.ds(b0, CHUNK)], x_v)
            gather.wait()

            def row_body(r, rcarry):
                for j in range(DIM // LANES):
                    sl = pl.ds(j * LANES, LANES)
                    x_v[r, sl] = x_v[r, sl] * g_v[r, sl]
                return rcarry

            lax.fori_loop(0, CHUNK, row_body, 0)
            pltpu.sync_copy(x_v, out_hbm.at[pl.ds(b0, CHUNK)])
            return carry

        lax.fori_loop(0, NCHUNK, chunk_body, 0)

    return k(x, idx, gates)


def kernel(input, gate_index, gates):
    idx = gate_index.astype(jnp.int32).reshape(BATCH)
    return _sc_gate_mul(input, idx, gates)


# speedup vs baseline: 1.1274x; 1.1274x over previous
import functools

import jax
import jax.numpy as jnp
from jax import lax
from jax.experimental import pallas as pl
from jax.experimental.pallas import tpu as pltpu
from jax.experimental.pallas import tpu_sc as plsc

NUM_GATES = 1000
BATCH = 16384
DIM = 128
LANES = 16

NC = 2
NS = 16
NW = NC * NS

B_PER_W = BATCH // NW
CHUNK = 128
NCHUNK = B_PER_W // CHUNK
NBUF = 2


def _sc_gate_mul(x, idx, gates):
    mesh = plsc.VectorSubcoreMesh(core_axis_name="c", subcore_axis_name="s")

    @functools.partial(
        pl.kernel,
        mesh=mesh,
        out_type=jax.ShapeDtypeStruct((BATCH, DIM), jnp.float32),
        scratch_types=[
            pltpu.VMEM((NCHUNK, CHUNK), jnp.int32),
            pltpu.VMEM((NBUF, CHUNK, DIM), jnp.float32),
            pltpu.VMEM((NBUF, CHUNK, DIM), jnp.float32),
            pltpu.SemaphoreType.DMA((NBUF,)),
            pltpu.SemaphoreType.DMA((NBUF,)),
            pltpu.SemaphoreType.DMA((NBUF,)),
        ],
    )
    def k(x_hbm, idx_hbm, gates_hbm, out_hbm, idx_v, g_v, x_v,
          gsem, xsem, osem):
        wid = lax.axis_index("s") * NC + lax.axis_index("c")
        base = wid * B_PER_W

        pltpu.sync_copy(idx_hbm.at[wid], idx_v)

        def fetch(c):
            b = c % NBUF
            pltpu.async_copy(gates_hbm.at[idx_v.at[c]], g_v.at[b],
                             gsem.at[b])
            pltpu.async_copy(x_hbm.at[pl.ds(base + c * CHUNK, CHUNK)],
                             x_v.at[b], xsem.at[b])

        fetch(0)
        for c in range(NCHUNK):
            b = c % NBUF
            if c + 1 < NCHUNK:
                if c >= 1:
                    pltpu.make_async_copy(
                        x_v.at[(c + 1) % NBUF],
                        out_hbm.at[pl.ds(base + (c - 1) * CHUNK, CHUNK)],
                        osem.at[(c + 1) % NBUF],
                    ).wait()
                fetch(c + 1)
            pltpu.make_async_copy(gates_hbm.at[idx_v.at[c]], g_v.at[b],
                                  gsem.at[b]).wait()
            pltpu.make_async_copy(x_hbm.at[pl.ds(base + c * CHUNK, CHUNK)],
                                  x_v.at[b], xsem.at[b]).wait()

            def row_body(r, rcarry):
                for j in range(DIM // LANES):
                    sl = pl.ds(j * LANES, LANES)
                    x_v[b, r, sl] = x_v[b, r, sl] * g_v[b, r, sl]
                return rcarry

            lax.fori_loop(0, CHUNK, row_body, 0)
            pltpu.async_copy(x_v.at[b],
                             out_hbm.at[pl.ds(base + c * CHUNK, CHUNK)],
                             osem.at[b])

        for c in range(NCHUNK - NBUF, NCHUNK):
            b = c % NBUF
            pltpu.make_async_copy(
                x_v.at[b],
                out_hbm.at[pl.ds(base + c * CHUNK, CHUNK)],
                osem.at[b],
            ).wait()

    return k(x, idx, gates)


def kernel(input, gate_index, gates):
    idx = gate_index.astype(jnp.int32).reshape(NW, NCHUNK, CHUNK)
    return _sc_gate_mul(input, idx, gates)
